# Initial kernel scaffold; baseline (speedup 1.0000x reference)
import jax
import jax.numpy as jnp
from jax.experimental import pallas as pl


def kernel(x, edge_index, W_in, b_in, att_l, att_r, W_out, b_out):
    raise NotImplementedError("write your pallas kernel here")


# speedup vs baseline: 3.1416x; 3.1416x over previous
import jax
import jax.numpy as jnp
from jax import lax
from jax.experimental import pallas as pl
from jax.experimental.pallas import tpu as pltpu
from jax.experimental.pallas import tpu_sc as plsc

_N = 10000
_E = 160000
_IN = 256
_H = 512
_OUT = 256
_L = 4
_EPS = 0.1

_NP = 10240
_NC = 2
_NS = 16
_CW = 128
_NCH = _H // _CW
_GB = 128
_NBG = 84
_EPT = _NBG * _GB
_NPT = _NP // _NS
_DRB = 32

_R = 1024


def _tc_in_body(x_ref, w_ref, b_ref, h_ref):
    h = jnp.dot(x_ref[...], w_ref[...], preferred_element_type=jnp.float32)
    h = h + b_ref[...]
    for c in range(_NCH):
        h_ref[c] = h[:, c * _CW:(c + 1) * _CW]


def _tc_in(x, w, b):
    return pl.pallas_call(
        _tc_in_body,
        out_shape=jax.ShapeDtypeStruct((_NCH, _NP, _CW), jnp.float32),
        grid=(_NP // _R,),
        in_specs=[
            pl.BlockSpec((_R, _IN), lambda i: (i, 0)),
            pl.BlockSpec((_IN, _H), lambda i: (0, 0)),
            pl.BlockSpec((1, _H), lambda i: (0, 0)),
        ],
        out_specs=pl.BlockSpec((_NCH, _R, _CW), lambda i: (0, i, 0)),
    )(x, w, b)


def _tc_att_body(h_ref, a_ref, o_ref):
    acc = jnp.zeros((_R, 8), jnp.float32)
    for c in range(_NCH):
        acc = acc + jnp.dot(h_ref[c], a_ref[c],
                            preferred_element_type=jnp.float32)
    o_ref[...] = acc


def _tc_att(h, amat):
    return pl.pallas_call(
        _tc_att_body,
        out_shape=jax.ShapeDtypeStruct((_NP, 8), jnp.float32),
        grid=(_NP // _R,),
        in_specs=[
            pl.BlockSpec((_NCH, _R, _CW), lambda i: (0, i, 0)),
            pl.BlockSpec((_NCH, _CW, 8), lambda i: (0, 0, 0)),
        ],
        out_specs=pl.BlockSpec((_R, 8), lambda i: (i, 0)),
    )(h, amat)


def _tc_out_body(h_ref, w_ref, b_ref, y_ref):
    acc = b_ref[...] + jnp.zeros((_R, _OUT), jnp.float32)
    for c in range(_NCH):
        acc = acc + jnp.dot(h_ref[c], w_ref[pl.ds(c * _CW, _CW), :],
                            preferred_element_type=jnp.float32)
    y_ref[...] = acc


def _tc_out(h, w, b):
    return pl.pallas_call(
        _tc_out_body,
        out_shape=jax.ShapeDtypeStruct((_NP, _OUT), jnp.float32),
        grid=(_NP // _R,),
        in_specs=[
            pl.BlockSpec((_NCH, _R, _CW), lambda i: (0, i, 0)),
            pl.BlockSpec((_H, _OUT), lambda i: (0, 0)),
            pl.BlockSpec((1, _OUT), lambda i: (0, 0)),
        ],
        out_specs=pl.BlockSpec((_R, _OUT), lambda i: (i, 0)),
    )(h, w, b)


def _sc_body(h_ref, h0_ref, al_ref, ar_ref, srcs_ref, dsts_ref, nrms_ref,
             out_ref, acc, srcv, dstv, ubuf, arbuf, nrmbuf, rows, work):
    cid = lax.axis_index("c")
    sid = lax.axis_index("s")

    pltpu.sync_copy(srcs_ref.at[sid], srcv)
    pltpu.sync_copy(dsts_ref.at[sid], dstv)

    zero16 = jnp.zeros((16,), jnp.float32)
    for j in range(2):
        chunk = cid * 2 + j

        def z_body(r, _):
            for k in range(8):
                work[r, pl.ds(k * 16, 16)] = zero16
            return 0

        lax.fori_loop(0, _DRB, z_body, 0)
        for k in range(20):
            pltpu.sync_copy(
                work, acc.at[pl.ds(sid * _NPT + k * _DRB, _DRB)])
        plsc.subcore_barrier()

        def e_body(b, _):
            pltpu.sync_copy(al_ref.at[srcv.at[pl.ds(b * _GB, _GB)]], ubuf)
            pltpu.sync_copy(ar_ref.at[dstv.at[b]], arbuf)
            pltpu.sync_copy(nrms_ref.at[sid].at[pl.ds(b * _GB, _GB)], nrmbuf)
            for jj in range(8):
                sl = pl.ds(jj * 16, 16)
                u = ubuf[sl] + arbuf[sl]
                ex = jnp.exp(-2.0 * jnp.abs(u))
                t = (1.0 - ex) / (1.0 + ex)
                t = jnp.where(u < 0.0, -t, t)
                ubuf[sl] = nrmbuf[sl] * t
            pltpu.sync_copy(
                h_ref.at[chunk].at[srcv.at[pl.ds(b * _GB, _GB)]], rows)

            def s_body(e, _):
                cv = plsc.load_gather(ubuf, [jnp.full((16,), e, jnp.int32)])
                for k in range(8):
                    sl = pl.ds(k * 16, 16)
                    rows[e, sl] = rows[e, sl] * cv
                return 0

            lax.fori_loop(0, _GB, s_body, 0)
            pltpu.sync_copy(rows, acc.at[dstv.at[b]], add=True)
            return 0

        lax.fori_loop(0, _NBG, e_body, 0)
        plsc.subcore_barrier()

        for k in range(20):
            r0 = sid * _NPT + k * _DRB
            pltpu.sync_copy(acc.at[pl.ds(r0, _DRB)], work)
            pltpu.sync_copy(h0_ref.at[chunk].at[pl.ds(r0, _DRB)],
                            rows.at[pl.ds(0, _DRB)])

            def d_body(r, _):
                for kk in range(8):
                    sl = pl.ds(kk * 16, 16)
                    work[r, sl] = work[r, sl] + _EPS * rows[r, sl]
                return 0

            lax.fori_loop(0, _DRB, d_body, 0)
            pltpu.sync_copy(work, out_ref.at[chunk].at[pl.ds(r0, _DRB)])


def _sc_layer(h, h0, al, ar, srcs, dsts, nrms):
    mesh = plsc.VectorSubcoreMesh(core_axis_name="c", subcore_axis_name="s",
                                  num_cores=_NC, num_subcores=_NS)
    kern = pl.kernel(
        _sc_body,
        out_type=jax.ShapeDtypeStruct((_NCH, _NP, _CW), jnp.float32),
        mesh=mesh,
        compiler_params=pltpu.CompilerParams(needs_layout_passes=False),
        scratch_types=[
            pltpu.VMEM_SHARED((_NP, _CW), jnp.float32),
            pltpu.VMEM((_EPT,), jnp.int32),
            pltpu.VMEM((_NBG, _GB), jnp.int32),
            pltpu.VMEM((_GB,), jnp.float32),
            pltpu.VMEM((_GB,), jnp.float32),
            pltpu.VMEM((_GB,), jnp.float32),
            pltpu.VMEM((_GB, _CW), jnp.float32),
            pltpu.VMEM((_DRB, _CW), jnp.float32),
        ],
    )
    return kern(h, h0, al, ar, srcs, dsts, nrms)


def kernel(x, edge_index, W_in, b_in, att_l, att_r, W_out, b_out):
    src, dst = edge_index[0], edge_index[1]
    loop = jnp.arange(_N, dtype=src.dtype)
    src = jnp.concatenate([src, loop])
    dst = jnp.concatenate([dst, loop])
    deg = jax.ops.segment_sum(jnp.ones(src.shape[0], jnp.float32), dst,
                              num_segments=_N)
    dinv = jnp.where(deg > 0, lax.rsqrt(deg), 0.0)
    norm = dinv[src] * dinv[dst]

    pad = _NS * _EPT - src.shape[0]
    srcp = jnp.concatenate([src, jnp.zeros((pad,), src.dtype)])
    dstp = jnp.concatenate([dst, jnp.zeros((pad,), dst.dtype)])
    nrmp = jnp.concatenate([norm, jnp.zeros((pad,), jnp.float32)])
    srcp = srcp.reshape(_NS, _EPT)
    dstp = dstp.reshape(_NS, _NBG, _GB)
    nrmp = nrmp.reshape(_NS, _EPT)

    xp = jnp.pad(x, ((0, _NP - _N), (0, 0)))
    h0 = _tc_in(xp, W_in, b_in.reshape(1, _H))
    h = h0
    for l in range(_L):
        amat = jnp.stack([att_l[l].reshape(_NCH, _CW),
                          att_r[l].reshape(_NCH, _CW)], axis=-1)
        amat = jnp.pad(amat, ((0, 0), (0, 0), (0, 6)))
        alar = _tc_att(h, amat)
        h = _sc_layer(h, h0, alar[:, 0], alar[:, 1], srcp, dstp, nrmp)
    return _tc_out(h, W_out, b_out.reshape(1, _OUT))[:_N]
